# baseline (speedup 1.0000x reference)
import functools

import jax
import jax.numpy as jnp
from jax import lax
from jax.experimental import pallas as pl
from jax.experimental.pallas import tpu as pltpu, tpu_sc as plsc

NJ = 50000
NW = 64
D = 16
E = 800000
EP = 200000

NJP = 51200
ROWS_PER_TILE = NJP // 16
BLK = 2048
GRID = NJP // BLK
CH = 128
DUMMY = NJ

NC = 2
NS = 16


def _pad_len(n_edges_per_sc):
    per_tile = -(-n_edges_per_sc // (NS * CH))
    return per_tile * NS * CH, per_tile


def _make_segsum(n_ch, table_rows):
    mesh = plsc.VectorSubcoreMesh(core_axis_name="c", subcore_axis_name="s")
    per_sc = n_ch * NS * CH

    @functools.partial(
        pl.kernel,
        out_type=jax.ShapeDtypeStruct((NC, NJP, D), jnp.float32),
        mesh=mesh,
        scratch_types=[
            pltpu.VMEM_SHARED((NJP, D), jnp.float32),
            pltpu.VMEM((CH,), jnp.int32),
            pltpu.VMEM((1, CH), jnp.int32),
            pltpu.VMEM((CH, D), jnp.float32),
            pltpu.SemaphoreType.DMA,
        ],
        compiler_params=pltpu.CompilerParams(use_tc_tiling_on_sc=False),
    )
    def segsum(table0, table1, src, dst, zrows, out, acc, idx_s, idx_d, rows, sem):
        c = lax.axis_index("c")
        s = lax.axis_index("s")
        pltpu.sync_copy(zrows, acc.at[pl.ds(s * ROWS_PER_TILE, ROWS_PER_TILE)])
        plsc.subcore_barrier()

        base = s * n_ch

        def run(table):
            def body(j, _):
                pltpu.sync_copy(src.at[c, pl.ds((base + j) * CH, CH)], idx_s)
                pltpu.sync_copy(dst.at[c, base + j], idx_d.at[0])
                pltpu.async_copy(table.at[idx_s], rows, sem).wait()
                pltpu.sync_copy(rows, acc.at[idx_d.at[0]], add=True)
                return 0

            lax.fori_loop(0, n_ch, body, 0)

        @pl.when(c == 0)
        def _():
            run(table0)

        @pl.when(c == 1)
        def _():
            run(table1)

        plsc.subcore_barrier()
        pltpu.sync_copy(acc.at[pl.ds(s * ROWS_PER_TILE, ROWS_PER_TILE)],
                        out.at[c, pl.ds(s * ROWS_PER_TILE, ROWS_PER_TILE)])

    def call(table0, table1, src2, dst2, zrows):
        assert src2.shape == (NC, per_sc) and table0.shape == (table_rows, D)
        return segsum(table0, table1, src2, dst2.reshape(NC, per_sc // CH, CH), zrows)

    return call


def _embed_body(x, w8, bj, xw, ww, bw, d0, d1, wp, wn, hj, yp, yn, hww, deg):
    h = jnp.dot(x[...], w8[...], preferred_element_type=jnp.float32) + bj[...]
    dg = d0[...] + d1[...]
    hj[...] = h
    deg[...] = dg
    yp[...] = jnp.dot(h * lax.rsqrt(jnp.maximum(dg[:, 0:1], 1.0)), wp[...],
                      preferred_element_type=jnp.float32)
    yn[...] = jnp.dot(h * lax.rsqrt(jnp.maximum(dg[:, 2:3], 1.0)), wn[...],
                      preferred_element_type=jnp.float32)
    hww[...] = jnp.dot(xw[...], ww[...], preferred_element_type=jnp.float32) + bw[...]


def _round_core(hj, aggp, aggn, sg0, sg1, deg, ws, wsn, bp, bn, bs):
    dg = deg[...]
    sip = lax.rsqrt(jnp.maximum(dg[:, 1:2], 1.0))
    sin = lax.rsqrt(jnp.maximum(dg[:, 3:4], 1.0))
    dpr = 1.0 / jnp.maximum(dg[:, 4:5], 1.0)
    sage = (sg0[...] + sg1[...]) * dpr
    return (aggp[...] * sip + aggn[...] * sin
            + jnp.dot(hj[...], ws[...], preferred_element_type=jnp.float32)
            + jnp.dot(sage, wsn[...], preferred_element_type=jnp.float32)
            + bp[...] + bn[...] + bs[...])


def _comb1_body(hj, aggp, aggn, sg0, sg1, deg, ws, wsn, bp, bn, bs, wp, wn,
                hj1, yp, yn):
    h = _round_core(hj, aggp, aggn, sg0, sg1, deg, ws, wsn, bp, bn, bs)
    dg = deg[...]
    hj1[...] = h
    yp[...] = jnp.dot(h * lax.rsqrt(jnp.maximum(dg[:, 0:1], 1.0)), wp[...],
                      preferred_element_type=jnp.float32)
    yn[...] = jnp.dot(h * lax.rsqrt(jnp.maximum(dg[:, 2:3], 1.0)), wn[...],
                      preferred_element_type=jnp.float32)


def _comb2_body(hj, aggp, aggn, sg0, sg1, deg, ws, wsn, bp, bn, bs, hw,
                res):
    h = _round_core(hj, aggp, aggn, sg0, sg1, deg, ws, wsn, bp, bn, bs)
    res[...] = lax.dot_general(h, hw[...], (((1,), (1,)), ((), ())),
                               preferred_element_type=jnp.float32)


def _row_spec(w=D):
    return pl.BlockSpec((BLK, w), lambda i: (i, 0))


def _const_spec(shape):
    return pl.BlockSpec(shape, lambda i: (0,) * len(shape))


def kernel(x_job, x_worker, edge_precede, edge_next, proc_src, proc_dst,
           W_job, b_job, W_worker, b_worker, W_precede, b_precede,
           W_next, b_next, W_sage_self, W_sage_neigh, b_sage):
    f32, i32 = jnp.float32, jnp.int32

    zrows = jnp.zeros((ROWS_PER_TILE, D), f32)

    gc_per_sc, gc_nch = _pad_len(E)
    padg = gc_per_sc - E

    def pad_edges(src, dst, n):
        return (jnp.concatenate([src, jnp.zeros((n,), i32)]),
                jnp.concatenate([dst, jnp.full((n,), DUMMY, i32)]))

    sp, dp = pad_edges(edge_precede[0], edge_precede[1], padg)
    sn, dn = pad_edges(edge_next[0], edge_next[1], padg)
    gc_src = jnp.stack([sp, sn])
    gc_dst = jnp.stack([dp, dn])

    sg_total, _ = _pad_len(EP // NC)
    sg_nch = sg_total // (NS * CH)
    ssrc, sdst = pad_edges(proc_src, proc_dst, NC * sg_total - EP)
    sg_src = ssrc.reshape(NC, sg_total)
    sg_dst = sdst.reshape(NC, sg_total)

    ET = 4 * E + EP
    dg_per_sc, dg_nch = _pad_len(-(-ET // NC))
    padd = NC * dg_per_sc - ET
    dg_src = jnp.concatenate([
        jnp.full((E,), 0, i32), jnp.full((E,), 1, i32),
        jnp.full((E,), 2, i32), jnp.full((E,), 3, i32),
        jnp.full((EP,), 4, i32), jnp.full((padd,), 7, i32),
    ]).reshape(NC, dg_per_sc)
    dg_dst = jnp.concatenate([
        edge_precede[0], edge_precede[1], edge_next[0], edge_next[1],
        proc_dst, jnp.full((padd,), DUMMY, i32),
    ]).reshape(NC, dg_per_sc)
    onehot = jnp.zeros((8, D), f32).at[jnp.arange(5), jnp.arange(5)].set(1.0)

    xjp = jnp.pad(x_job, ((0, NJP - NJ), (0, 1)))
    w8 = jnp.pad(W_job, ((0, 1), (0, 0)))
    xwp = jnp.pad(x_worker, ((0, 0), (0, 1)))
    ww4 = jnp.pad(W_worker, ((0, 1), (0, 0)))
    b2 = lambda b: b.reshape(1, D)

    seg_deg = _make_segsum(dg_nch, 8)
    seg_sage = _make_segsum(sg_nch, NW)
    seg_gc = _make_segsum(gc_nch, NJP)

    deg2 = seg_deg(onehot, onehot, dg_src, dg_dst, zrows)

    embed = pl.pallas_call(
        _embed_body,
        grid=(GRID,),
        in_specs=[_row_spec(8), _const_spec((8, D)), _const_spec((1, D)),
                  _const_spec((NW, 4)), _const_spec((4, D)), _const_spec((1, D)),
                  _row_spec(), _row_spec(),
                  _const_spec((D, D)), _const_spec((D, D))],
        out_specs=[_row_spec(), _row_spec(), _row_spec(),
                   _const_spec((NW, D)), _row_spec()],
        out_shape=[jax.ShapeDtypeStruct((NJP, D), f32)] * 3
        + [jax.ShapeDtypeStruct((NW, D), f32), jax.ShapeDtypeStruct((NJP, D), f32)],
    )
    hj0, yp1, yn1, hw, deg = embed(xjp, w8, b2(b_job), xwp, ww4, b2(b_worker),
                                   deg2[0], deg2[1], W_precede, W_next)

    sage2 = seg_sage(hw, hw, sg_src, sg_dst, zrows)

    agg1 = seg_gc(yp1, yn1, gc_src, gc_dst, zrows)

    comb_in_specs = [_row_spec()] * 6 + [_const_spec((D, D))] * 2 + [_const_spec((1, D))] * 3
    comb1 = pl.pallas_call(
        _comb1_body,
        grid=(GRID,),
        in_specs=comb_in_specs + [_const_spec((D, D))] * 2,
        out_specs=[_row_spec()] * 3,
        out_shape=[jax.ShapeDtypeStruct((NJP, D), f32)] * 3,
    )
    hj1, yp2, yn2 = comb1(hj0, agg1[0], agg1[1], sage2[0], sage2[1], deg,
                          W_sage_self, W_sage_neigh,
                          b2(b_precede), b2(b_next), b2(b_sage),
                          W_precede, W_next)

    agg2 = seg_gc(yp2, yn2, gc_src, gc_dst, zrows)

    comb2 = pl.pallas_call(
        _comb2_body,
        grid=(GRID,),
        in_specs=comb_in_specs + [_const_spec((NW, D))],
        out_specs=pl.BlockSpec((BLK, NW), lambda i: (i, 0)),
        out_shape=jax.ShapeDtypeStruct((NJP, NW), f32),
    )
    res = comb2(hj1, agg2[0], agg2[1], sage2[0], sage2[1], deg,
                W_sage_self, W_sage_neigh,
                b2(b_precede), b2(b_next), b2(b_sage), hw)
    return res[:NJ]


# speedup vs baseline: 1.0082x; 1.0082x over previous
import functools

import jax
import jax.numpy as jnp
from jax import lax
from jax.experimental import pallas as pl
from jax.experimental.pallas import tpu as pltpu, tpu_sc as plsc

NJ = 50000
NW = 64
D = 16
E = 800000
EP = 200000

NJP = 51200
ROWS_PER_TILE = NJP // 16
BLK = 2048
GRID = NJP // BLK
CH = 128
DUMMY = NJ

NC = 2
NS = 16


IBC = 8
BLK_E = IBC * CH


def _pad_len(n_edges_per_sc):
    per_tile = -(-n_edges_per_sc // (NS * CH))
    per_tile = -(-per_tile // (4 * IBC)) * (4 * IBC)
    return per_tile * NS * CH, per_tile


def _make_segsum(n_ch, table_rows):
    mesh = plsc.VectorSubcoreMesh(core_axis_name="c", subcore_axis_name="s")
    per_sc = n_ch * NS * CH

    n_blk = n_ch // IBC
    nb4 = n_blk // 4
    assert n_blk % 4 == 0 and nb4 >= 1

    @functools.partial(
        pl.kernel,
        out_type=jax.ShapeDtypeStruct((NC, NJP, D), jnp.float32),
        mesh=mesh,
        scratch_types=[
            pltpu.VMEM_SHARED((NJP, D), jnp.float32),
            pltpu.VMEM((4, BLK_E), jnp.int32),
            pltpu.VMEM((4, IBC, CH), jnp.int32),
            pltpu.VMEM((2, BLK_E, D), jnp.float32),
        ] + [pltpu.SemaphoreType.DMA] * 8,
        compiler_params=pltpu.CompilerParams(use_tc_tiling_on_sc=False),
    )
    def segsum(table0, table1, src, dst, zrows, out, acc, idxs, idxd, rows,
               i0, i1, i2, i3, g0, g1, s0, s1):
        c = lax.axis_index("c")
        s = lax.axis_index("s")
        isem = [i0, i1, i2, i3]
        gsem = [g0, g1]
        ssem = [s0, s1]
        pltpu.sync_copy(zrows, acc.at[pl.ds(s * ROWS_PER_TILE, ROWS_PER_TILE)])
        plsc.subcore_barrier()

        cbase = s * n_ch

        def issue_idx(b, u):
            pltpu.async_copy(src.at[c, pl.ds((cbase + b * IBC) * CH, BLK_E)],
                             idxs.at[u], isem[u])
            pltpu.async_copy(dst.at[c, pl.ds(cbase + b * IBC, IBC)],
                             idxd.at[u], isem[u])

        def wait_idx(b, u):
            pltpu.make_async_copy(src.at[c, pl.ds((cbase + b * IBC) * CH, BLK_E)],
                                  idxs.at[u], isem[u]).wait()
            pltpu.make_async_copy(dst.at[c, pl.ds(cbase + b * IBC, IBC)],
                                  idxd.at[u], isem[u]).wait()

        def drain_rows(table, sem):
            pltpu.make_async_copy(table.at[pl.ds(0, BLK_E)],
                                  acc.at[pl.ds(0, BLK_E)], sem).wait()

        def run(table):
            def blk4(b4, _):
                for u in range(4):
                    b = b4 * 4 + u
                    sl = u % 2

                    if u < 2:
                        @pl.when(b4 > 0)
                        def _():
                            drain_rows(table, ssem[sl])

                        issue_idx(b + 2, (u + 2) % 4)
                    else:
                        drain_rows(table, ssem[sl])

                        @pl.when(b4 < nb4 - 1)
                        def _():
                            issue_idx(b + 2, (u + 2) % 4)

                    wait_idx(b, u)
                    for k in range(IBC):
                        pltpu.async_copy(
                            table.at[idxs.at[u].at[pl.ds(k * CH, CH)]],
                            rows.at[sl].at[pl.ds(k * CH, CH)], gsem[sl])
                    drain_rows(table, gsem[sl])
                    for k in range(IBC):
                        pltpu.async_copy(rows.at[sl].at[pl.ds(k * CH, CH)],
                                         acc.at[idxd.at[u, k]], ssem[sl],
                                         add=True)
                return 0

            issue_idx(0, 0)
            issue_idx(1, 1)
            lax.fori_loop(0, nb4, blk4, 0)
            drain_rows(table, ssem[0])
            drain_rows(table, ssem[1])

        @pl.when(c == 0)
        def _():
            run(table0)

        @pl.when(c == 1)
        def _():
            run(table1)

        plsc.subcore_barrier()
        pltpu.sync_copy(acc.at[pl.ds(s * ROWS_PER_TILE, ROWS_PER_TILE)],
                        out.at[c, pl.ds(s * ROWS_PER_TILE, ROWS_PER_TILE)])

    def call(table0, table1, src2, dst2, zrows):
        assert src2.shape == (NC, per_sc) and table0.shape == (table_rows, D)
        return segsum(table0, table1, src2, dst2.reshape(NC, per_sc // CH, CH), zrows)

    return call


def _embed_body(x, w8, bj, xw, ww, bw, d0, d1, wp, wn, hj, yp, yn, hww, deg):
    h = jnp.dot(x[...], w8[...], preferred_element_type=jnp.float32) + bj[...]
    dg = d0[...] + d1[...]
    hj[...] = h
    deg[...] = dg
    yp[...] = jnp.dot(h * lax.rsqrt(jnp.maximum(dg[:, 0:1], 1.0)), wp[...],
                      preferred_element_type=jnp.float32)
    yn[...] = jnp.dot(h * lax.rsqrt(jnp.maximum(dg[:, 2:3], 1.0)), wn[...],
                      preferred_element_type=jnp.float32)
    hww[...] = jnp.dot(xw[...], ww[...], preferred_element_type=jnp.float32) + bw[...]


def _round_core(hj, aggp, aggn, sg0, sg1, deg, ws, wsn, bp, bn, bs):
    dg = deg[...]
    sip = lax.rsqrt(jnp.maximum(dg[:, 1:2], 1.0))
    sin = lax.rsqrt(jnp.maximum(dg[:, 3:4], 1.0))
    dpr = 1.0 / jnp.maximum(dg[:, 4:5], 1.0)
    sage = (sg0[...] + sg1[...]) * dpr
    return (aggp[...] * sip + aggn[...] * sin
            + jnp.dot(hj[...], ws[...], preferred_element_type=jnp.float32)
            + jnp.dot(sage, wsn[...], preferred_element_type=jnp.float32)
            + bp[...] + bn[...] + bs[...])


def _comb1_body(hj, aggp, aggn, sg0, sg1, deg, ws, wsn, bp, bn, bs, wp, wn,
                hj1, yp, yn):
    h = _round_core(hj, aggp, aggn, sg0, sg1, deg, ws, wsn, bp, bn, bs)
    dg = deg[...]
    hj1[...] = h
    yp[...] = jnp.dot(h * lax.rsqrt(jnp.maximum(dg[:, 0:1], 1.0)), wp[...],
                      preferred_element_type=jnp.float32)
    yn[...] = jnp.dot(h * lax.rsqrt(jnp.maximum(dg[:, 2:3], 1.0)), wn[...],
                      preferred_element_type=jnp.float32)


def _comb2_body(hj, aggp, aggn, sg0, sg1, deg, ws, wsn, bp, bn, bs, hw,
                res):
    h = _round_core(hj, aggp, aggn, sg0, sg1, deg, ws, wsn, bp, bn, bs)
    res[...] = lax.dot_general(h, hw[...], (((1,), (1,)), ((), ())),
                               preferred_element_type=jnp.float32)


def _row_spec(w=D):
    return pl.BlockSpec((BLK, w), lambda i: (i, 0))


def _const_spec(shape):
    return pl.BlockSpec(shape, lambda i: (0,) * len(shape))


def kernel(x_job, x_worker, edge_precede, edge_next, proc_src, proc_dst,
           W_job, b_job, W_worker, b_worker, W_precede, b_precede,
           W_next, b_next, W_sage_self, W_sage_neigh, b_sage):
    f32, i32 = jnp.float32, jnp.int32

    zrows = jnp.zeros((ROWS_PER_TILE, D), f32)

    gc_per_sc, gc_nch = _pad_len(E)
    padg = gc_per_sc - E

    def pad_edges(src, dst, n):
        return (jnp.concatenate([src, jnp.zeros((n,), i32)]),
                jnp.concatenate([dst, jnp.full((n,), DUMMY, i32)]))

    sp, dp = pad_edges(edge_precede[0], edge_precede[1], padg)
    sn, dn = pad_edges(edge_next[0], edge_next[1], padg)
    gc_src = jnp.stack([sp, sn])
    gc_dst = jnp.stack([dp, dn])

    sg_total, _ = _pad_len(EP // NC)
    sg_nch = sg_total // (NS * CH)
    ssrc, sdst = pad_edges(proc_src, proc_dst, NC * sg_total - EP)
    sg_src = ssrc.reshape(NC, sg_total)
    sg_dst = sdst.reshape(NC, sg_total)

    ET = 4 * E + EP
    dg_per_sc, dg_nch = _pad_len(-(-ET // NC))
    padd = NC * dg_per_sc - ET
    dg_src = jnp.concatenate([
        jnp.full((E,), 0, i32), jnp.full((E,), 1, i32),
        jnp.full((E,), 2, i32), jnp.full((E,), 3, i32),
        jnp.full((EP,), 4, i32), jnp.full((padd,), 7, i32),
    ]).reshape(NC, dg_per_sc)
    dg_dst = jnp.concatenate([
        edge_precede[0], edge_precede[1], edge_next[0], edge_next[1],
        proc_dst, jnp.full((padd,), DUMMY, i32),
    ]).reshape(NC, dg_per_sc)
    onehot = jnp.zeros((8, D), f32).at[jnp.arange(5), jnp.arange(5)].set(1.0)

    xjp = jnp.pad(x_job, ((0, NJP - NJ), (0, 1)))
    w8 = jnp.pad(W_job, ((0, 1), (0, 0)))
    xwp = jnp.pad(x_worker, ((0, 0), (0, 1)))
    ww4 = jnp.pad(W_worker, ((0, 1), (0, 0)))
    b2 = lambda b: b.reshape(1, D)

    seg_deg = _make_segsum(dg_nch, 8)
    seg_sage = _make_segsum(sg_nch, NW)
    seg_gc = _make_segsum(gc_nch, NJP)

    deg2 = seg_deg(onehot, onehot, dg_src, dg_dst, zrows)

    embed = pl.pallas_call(
        _embed_body,
        grid=(GRID,),
        in_specs=[_row_spec(8), _const_spec((8, D)), _const_spec((1, D)),
                  _const_spec((NW, 4)), _const_spec((4, D)), _const_spec((1, D)),
                  _row_spec(), _row_spec(),
                  _const_spec((D, D)), _const_spec((D, D))],
        out_specs=[_row_spec(), _row_spec(), _row_spec(),
                   _const_spec((NW, D)), _row_spec()],
        out_shape=[jax.ShapeDtypeStruct((NJP, D), f32)] * 3
        + [jax.ShapeDtypeStruct((NW, D), f32), jax.ShapeDtypeStruct((NJP, D), f32)],
    )
    hj0, yp1, yn1, hw, deg = embed(xjp, w8, b2(b_job), xwp, ww4, b2(b_worker),
                                   deg2[0], deg2[1], W_precede, W_next)

    sage2 = seg_sage(hw, hw, sg_src, sg_dst, zrows)

    agg1 = seg_gc(yp1, yn1, gc_src, gc_dst, zrows)

    comb_in_specs = [_row_spec()] * 6 + [_const_spec((D, D))] * 2 + [_const_spec((1, D))] * 3
    comb1 = pl.pallas_call(
        _comb1_body,
        grid=(GRID,),
        in_specs=comb_in_specs + [_const_spec((D, D))] * 2,
        out_specs=[_row_spec()] * 3,
        out_shape=[jax.ShapeDtypeStruct((NJP, D), f32)] * 3,
    )
    hj1, yp2, yn2 = comb1(hj0, agg1[0], agg1[1], sage2[0], sage2[1], deg,
                          W_sage_self, W_sage_neigh,
                          b2(b_precede), b2(b_next), b2(b_sage),
                          W_precede, W_next)

    agg2 = seg_gc(yp2, yn2, gc_src, gc_dst, zrows)

    comb2 = pl.pallas_call(
        _comb2_body,
        grid=(GRID,),
        in_specs=comb_in_specs + [_const_spec((NW, D))],
        out_specs=pl.BlockSpec((BLK, NW), lambda i: (i, 0)),
        out_shape=jax.ShapeDtypeStruct((NJP, NW), f32),
    )
    res = comb2(hj1, agg2[0], agg2[1], sage2[0], sage2[1], deg,
                W_sage_self, W_sage_neigh,
                b2(b_precede), b2(b_next), b2(b_sage), hw)
    return res[:NJ]


# speedup vs baseline: 9.0767x; 9.0033x over previous
import functools

import jax
import jax.numpy as jnp
from jax import lax
from jax.experimental import pallas as pl
from jax.experimental.pallas import tpu as pltpu, tpu_sc as plsc

NJ = 50000
NW = 64
D = 16
E = 800000
EP = 200000

NJP = 51200
ROWS_PER_TILE = NJP // 16
BLK = 2048
GRID = NJP // BLK
CH = 128
DUMMY = NJ

NC = 2
NS = 16


IBC = 8
BLK_E = IBC * CH


def _pad_len(n_edges_per_sc):
    per_tile = -(-n_edges_per_sc // (NS * CH))
    per_tile = -(-per_tile // (4 * IBC)) * (4 * IBC)
    return per_tile * NS * CH, per_tile


HB = 2048
HROWS = ROWS_PER_TILE


def _make_deg_hist(groups, n_pers):
    mesh = plsc.VectorSubcoreMesh(core_axis_name="c", subcore_axis_name="s")
    max_slots = 3

    @functools.partial(
        pl.kernel,
        out_type=jax.ShapeDtypeStruct((5, HROWS, D), jnp.int32),
        mesh=mesh,
        scratch_types=[
            pltpu.VMEM_SHARED((max_slots * HROWS, D), jnp.int32),
            pltpu.VMEM((HROWS, D), jnp.int32),
            pltpu.VMEM((2, HB), jnp.int32),
            pltpu.VMEM((HROWS // CH, CH), jnp.int32),
            pltpu.SemaphoreType.DMA,
            pltpu.SemaphoreType.DMA,
        ],
        compiler_params=pltpu.CompilerParams(use_tc_tiling_on_sc=False,
                                             needs_layout_passes=False),
    )
    def deg(e0, e1, e2, e3, e4, zrows_i, iota3, out, acc, hist, ib, idq,
            isem, ssem):
        c = lax.axis_index("c")
        s = lax.axis_index("s")
        edges = [e0, e1, e2, e3, e4]
        ones = jnp.ones((D,), jnp.int32)

        zslice = max_slots * HROWS // NS
        pltpu.sync_copy(zrows_i.at[pl.ds(0, zslice)],
                        acc.at[pl.ds(s * zslice, zslice)])
        pltpu.sync_copy(zrows_i, hist)

        for t, (sc, s_lo, s_hi, slot) in enumerate(groups):
            n_per = n_pers[t]
            nb = n_per // HB
            assert nb % 2 == 0

            @pl.when((c == sc) & (s >= s_lo) & (s < s_hi))
            def _(t=t, s_lo=s_lo, slot=slot, nb=nb):
                p = s - s_lo
                ev = edges[t]
                pltpu.sync_copy(iota3.at[slot], idq)
                pltpu.async_copy(ev.at[p, pl.ds(0, HB)], ib.at[0], isem)

                def blk2(j2, _):
                    for sl in range(2):
                        j = j2 * 2 + sl
                        pltpu.make_async_copy(ev.at[p, pl.ds(j * HB, HB)],
                                              ib.at[sl], isem).wait()

                        @pl.when(j < nb - 1)
                        def _():
                            pltpu.async_copy(ev.at[p, pl.ds((j + 1) * HB, HB)],
                                             ib.at[1 - sl], isem)

                        def vec(k, _, sl=sl):
                            v = ib[sl, pl.ds(k * D, D)]
                            plsc.addupdate_scatter(hist, [v >> 4, v & 15], ones)
                            return 0

                        lax.fori_loop(0, HB // D, vec, 0)
                    return 0

                lax.fori_loop(0, nb // 2, blk2, 0)

        plsc.subcore_barrier()

        for t, (sc, s_lo, s_hi, slot) in enumerate(groups):
            @pl.when((c == sc) & (s >= s_lo) & (s < s_hi))
            def _(t=t):
                for j in range(HROWS // CH):
                    pltpu.async_copy(hist.at[pl.ds(j * CH, CH)],
                                     acc.at[idq.at[j]], ssem, add=True)
                pltpu.make_async_copy(hist, acc.at[pl.ds(0, HROWS)], ssem).wait()

        plsc.subcore_barrier()
        for t, (sc, s_lo, s_hi, slot) in enumerate(groups):
            @pl.when((c == sc) & (s == s_lo))
            def _(t=t, slot=slot):
                pltpu.sync_copy(acc.at[pl.ds(slot * HROWS, HROWS)], out.at[t])

    return deg


def _make_segsum(n_ch, table_rows, stage=False):
    mesh = plsc.VectorSubcoreMesh(core_axis_name="c", subcore_axis_name="s")
    per_sc = n_ch * NS * CH

    n_blk = n_ch // IBC
    nb4 = n_blk // 4
    assert n_blk % 4 == 0 and nb4 >= 1

    @functools.partial(
        pl.kernel,
        out_type=jax.ShapeDtypeStruct((NC, NJP, D), jnp.float32),
        mesh=mesh,
        scratch_types=[
            pltpu.VMEM_SHARED((NJP, D), jnp.float32),
            pltpu.VMEM((4, IBC, CH), jnp.int32),
            pltpu.VMEM((4, IBC, CH), jnp.int32),
            pltpu.VMEM((2, BLK_E, D), jnp.float32),
            pltpu.VMEM_SHARED((table_rows if stage else 1, D), jnp.float32),
        ] + [pltpu.SemaphoreType.DMA] * 8,
        compiler_params=pltpu.CompilerParams(use_tc_tiling_on_sc=False),
    )
    def segsum(table0, table1, src, dst, zrows, out, acc, idxs, idxd, rows,
               tstage, i0, i1, i2, i3, g0, g1, s0, s1):
        c = lax.axis_index("c")
        s = lax.axis_index("s")
        isem = [i0, i1, i2, i3]
        gsem = [g0, g1]
        ssem = [s0, s1]
        pltpu.sync_copy(zrows, acc.at[pl.ds(s * ROWS_PER_TILE, ROWS_PER_TILE)])
        if stage:
            @pl.when((c == 0) & (s == 0))
            def _():
                pltpu.sync_copy(table0, tstage)

            @pl.when((c == 1) & (s == 0))
            def _():
                pltpu.sync_copy(table1, tstage)
        plsc.subcore_barrier()

        cbase = s * n_ch

        def issue_idx(b, u):
            pltpu.async_copy(src.at[c, pl.ds(cbase + b * IBC, IBC)],
                             idxs.at[u], isem[u])
            pltpu.async_copy(dst.at[c, pl.ds(cbase + b * IBC, IBC)],
                             idxd.at[u], isem[u])

        def wait_idx(b, u):
            pltpu.make_async_copy(src.at[c, pl.ds(cbase + b * IBC, IBC)],
                                  idxs.at[u], isem[u]).wait()
            pltpu.make_async_copy(dst.at[c, pl.ds(cbase + b * IBC, IBC)],
                                  idxd.at[u], isem[u]).wait()

        def run(gtab):
            def scat_desc(u, sl, k):
                return pltpu.make_async_copy(rows.at[sl].at[pl.ds(k * CH, CH)],
                                             acc.at[idxd.at[u, k]], ssem[sl])

            def drain_scat(u, sl):
                for k in range(IBC):
                    scat_desc(u, sl, k).wait()

            def blk4(b4, _):
                for u in range(4):
                    b = b4 * 4 + u
                    sl = u % 2
                    u2 = (u + 2) % 4

                    if u < 2:
                        @pl.when(b4 > 0)
                        def _():
                            drain_scat(u2, sl)

                        issue_idx(b + 2, u2)
                    else:
                        drain_scat(u2, sl)

                        @pl.when(b4 < nb4 - 1)
                        def _():
                            issue_idx(b + 2, u2)

                    wait_idx(b, u)
                    gds = []
                    for k in range(IBC):
                        gds.append(pltpu.async_copy(
                            gtab.at[idxs.at[u, k]],
                            rows.at[sl].at[pl.ds(k * CH, CH)], gsem[sl]))
                    for g in gds:
                        g.wait()
                    for k in range(IBC):
                        pltpu.async_copy(rows.at[sl].at[pl.ds(k * CH, CH)],
                                         acc.at[idxd.at[u, k]], ssem[sl],
                                         add=True)
                return 0

            issue_idx(0, 0)
            issue_idx(1, 1)
            lax.fori_loop(0, nb4, blk4, 0)
            drain_scat(2, 0)
            drain_scat(3, 1)

        @pl.when(c == 0)
        def _():
            run(tstage if stage else table0)

        @pl.when(c == 1)
        def _():
            run(tstage if stage else table1)

        plsc.subcore_barrier()
        pltpu.sync_copy(acc.at[pl.ds(s * ROWS_PER_TILE, ROWS_PER_TILE)],
                        out.at[c, pl.ds(s * ROWS_PER_TILE, ROWS_PER_TILE)])

    def call(table0, table1, src2, dst2, zrows):
        assert src2.shape == (NC, per_sc) and table0.shape == (table_rows, D)
        return segsum(table0, table1, src2.reshape(NC, per_sc // CH, CH),
                      dst2.reshape(NC, per_sc // CH, CH), zrows)

    return call


def _embed_body(x, w8, bj, xw, ww, bw, dop, don, wp, wn, hj, yp, yn, hww):
    h = jnp.dot(x[...], w8[...], preferred_element_type=jnp.float32) + bj[...]
    hj[...] = h
    yp[...] = jnp.dot(h * lax.rsqrt(jnp.maximum(dop[...], 1.0)), wp[...],
                      preferred_element_type=jnp.float32)
    yn[...] = jnp.dot(h * lax.rsqrt(jnp.maximum(don[...], 1.0)), wn[...],
                      preferred_element_type=jnp.float32)
    hww[...] = jnp.dot(xw[...], ww[...], preferred_element_type=jnp.float32) + bw[...]


def _round_core(hj, aggp, aggn, sg0, sg1, dip, din, dpc, ws, wsn, bp, bn, bs):
    sip = lax.rsqrt(jnp.maximum(dip[...], 1.0))
    sin = lax.rsqrt(jnp.maximum(din[...], 1.0))
    dpr = 1.0 / jnp.maximum(dpc[...], 1.0)
    sage = (sg0[...] + sg1[...]) * dpr
    return (aggp[...] * sip + aggn[...] * sin
            + jnp.dot(hj[...], ws[...], preferred_element_type=jnp.float32)
            + jnp.dot(sage, wsn[...], preferred_element_type=jnp.float32)
            + bp[...] + bn[...] + bs[...])


def _comb1_body(hj, aggp, aggn, sg0, sg1, dip, din, dpc, dop, don,
                ws, wsn, bp, bn, bs, wp, wn, hj1, yp, yn):
    h = _round_core(hj, aggp, aggn, sg0, sg1, dip, din, dpc, ws, wsn, bp, bn, bs)
    hj1[...] = h
    yp[...] = jnp.dot(h * lax.rsqrt(jnp.maximum(dop[...], 1.0)), wp[...],
                      preferred_element_type=jnp.float32)
    yn[...] = jnp.dot(h * lax.rsqrt(jnp.maximum(don[...], 1.0)), wn[...],
                      preferred_element_type=jnp.float32)


def _comb2_body(hj, aggp, aggn, sg0, sg1, dip, din, dpc,
                ws, wsn, bp, bn, bs, hw, res):
    h = _round_core(hj, aggp, aggn, sg0, sg1, dip, din, dpc, ws, wsn, bp, bn, bs)
    res[...] = lax.dot_general(h, hw[...], (((1,), (1,)), ((), ())),
                               preferred_element_type=jnp.float32)


def _row_spec(w=D):
    return pl.BlockSpec((BLK, w), lambda i: (i, 0))


def _const_spec(shape):
    return pl.BlockSpec(shape, lambda i: (0,) * len(shape))


def kernel(x_job, x_worker, edge_precede, edge_next, proc_src, proc_dst,
           W_job, b_job, W_worker, b_worker, W_precede, b_precede,
           W_next, b_next, W_sage_self, W_sage_neigh, b_sage):
    f32, i32 = jnp.float32, jnp.int32

    zrows = jnp.zeros((ROWS_PER_TILE, D), f32)

    gc_per_sc, gc_nch = _pad_len(E)
    padg = gc_per_sc - E

    def pad_edges(src, dst, n):
        return (jnp.concatenate([src, jnp.zeros((n,), i32)]),
                jnp.concatenate([dst, jnp.full((n,), DUMMY, i32)]))

    sp, dp = pad_edges(edge_precede[0], edge_precede[1], padg)
    sn, dn = pad_edges(edge_next[0], edge_next[1], padg)
    gc_src = jnp.stack([sp, sn])
    gc_dst = jnp.stack([dp, dn])

    sg_total, _ = _pad_len(EP // NC)
    sg_nch = sg_total // (NS * CH)
    ssrc, sdst = pad_edges(proc_src, proc_dst, NC * sg_total - EP)
    sg_src = ssrc.reshape(NC, sg_total)
    sg_dst = sdst.reshape(NC, sg_total)

    DG_GROUPS = [(0, 0, 6, 0), (0, 6, 12, 1), (1, 0, 8, 0), (1, 8, 16, 1),
                 (0, 12, 16, 2)]
    dg_arrays = [edge_precede[0], edge_precede[1], edge_next[0], edge_next[1],
                 proc_dst]
    dg_npers, dg_padded = [], []
    for t, arr in enumerate(dg_arrays):
        g = DG_GROUPS[t][2] - DG_GROUPS[t][1]
        n_per = -(--(-arr.shape[0] // g) // (2 * HB)) * (2 * HB)
        dg_npers.append(n_per)
        dg_padded.append(jnp.concatenate(
            [arr, jnp.full((g * n_per - arr.shape[0],), DUMMY, i32)]
        ).reshape(g, n_per))
    zrows_i = jnp.zeros((HROWS, D), i32)
    iota3 = (jnp.arange(HROWS, dtype=i32).reshape(HROWS // CH, CH)[None]
             + (jnp.arange(3, dtype=i32) * HROWS)[:, None, None])

    xjp = jnp.pad(x_job, ((0, NJP - NJ), (0, 1)))
    w8 = jnp.pad(W_job, ((0, 1), (0, 0)))
    xwp = jnp.pad(x_worker, ((0, 0), (0, 1)))
    ww4 = jnp.pad(W_worker, ((0, 1), (0, 0)))
    b2 = lambda b: b.reshape(1, D)

    deg_hist = _make_deg_hist(DG_GROUPS, dg_npers)
    seg_sage = _make_segsum(sg_nch, NW)
    seg_gc = _make_segsum(gc_nch, NJP)

    deg5 = deg_hist(*dg_padded, zrows_i, iota3)
    degf = deg5.reshape(5, NJP, 1).astype(f32)

    embed = pl.pallas_call(
        _embed_body,
        grid=(GRID,),
        in_specs=[_row_spec(8), _const_spec((8, D)), _const_spec((1, D)),
                  _const_spec((NW, 4)), _const_spec((4, D)), _const_spec((1, D)),
                  _row_spec(1), _row_spec(1),
                  _const_spec((D, D)), _const_spec((D, D))],
        out_specs=[_row_spec(), _row_spec(), _row_spec(),
                   _const_spec((NW, D))],
        out_shape=[jax.ShapeDtypeStruct((NJP, D), f32)] * 3
        + [jax.ShapeDtypeStruct((NW, D), f32)],
    )
    hj0, yp1, yn1, hw = embed(xjp, w8, b2(b_job), xwp, ww4, b2(b_worker),
                              degf[0], degf[2], W_precede, W_next)

    sage2 = seg_sage(hw, hw, sg_src, sg_dst, zrows)

    agg1 = seg_gc(yp1, yn1, gc_src, gc_dst, zrows)

    comb_in_specs = ([_row_spec()] * 5 + [_row_spec(1)] * 3
                     + [_const_spec((D, D))] * 2 + [_const_spec((1, D))] * 3)
    comb1 = pl.pallas_call(
        _comb1_body,
        grid=(GRID,),
        in_specs=([_row_spec()] * 5 + [_row_spec(1)] * 5
                  + [_const_spec((D, D))] * 2 + [_const_spec((1, D))] * 3
                  + [_const_spec((D, D))] * 2),
        out_specs=[_row_spec()] * 3,
        out_shape=[jax.ShapeDtypeStruct((NJP, D), f32)] * 3,
    )
    hj1, yp2, yn2 = comb1(hj0, agg1[0], agg1[1], sage2[0], sage2[1],
                          degf[1], degf[3], degf[4], degf[0], degf[2],
                          W_sage_self, W_sage_neigh,
                          b2(b_precede), b2(b_next), b2(b_sage),
                          W_precede, W_next)

    agg2 = seg_gc(yp2, yn2, gc_src, gc_dst, zrows)

    comb2 = pl.pallas_call(
        _comb2_body,
        grid=(GRID,),
        in_specs=comb_in_specs + [_const_spec((NW, D))],
        out_specs=pl.BlockSpec((BLK, NW), lambda i: (i, 0)),
        out_shape=jax.ShapeDtypeStruct((NJP, NW), f32),
    )
    res = comb2(hj1, agg2[0], agg2[1], sage2[0], sage2[1],
                degf[1], degf[3], degf[4],
                W_sage_self, W_sage_neigh,
                b2(b_precede), b2(b_next), b2(b_sage), hw)
    return res[:NJ]


# speedup vs baseline: 11.1968x; 1.2336x over previous
import functools

import jax
import jax.numpy as jnp
from jax import lax
from jax.experimental import pallas as pl
from jax.experimental.pallas import tpu as pltpu, tpu_sc as plsc

NJ = 50000
NW = 64
D = 16
E = 800000
EP = 200000

NJP = 51200
ROWS_PER_TILE = NJP // 16
BLK = 2048
GRID = NJP // BLK
CH = 128
DUMMY = NJ

NC = 2
NS = 16


IBC = 8
BLK_E = IBC * CH


def _pad_len(n_edges_per_sc):
    per_tile = -(-n_edges_per_sc // (NS * CH))
    per_tile = -(-per_tile // (4 * IBC)) * (4 * IBC)
    return per_tile * NS * CH, per_tile


HB = 2048
HROWS = ROWS_PER_TILE


def _make_deg_hist(groups, n_pers):
    mesh = plsc.VectorSubcoreMesh(core_axis_name="c", subcore_axis_name="s")
    max_slots = 3

    @functools.partial(
        pl.kernel,
        out_type=jax.ShapeDtypeStruct((5, HROWS, D), jnp.int32),
        mesh=mesh,
        scratch_types=[
            pltpu.VMEM_SHARED((max_slots * HROWS, D), jnp.int32),
            pltpu.VMEM((HROWS, D), jnp.int32),
            pltpu.VMEM((2, HB), jnp.int32),
            pltpu.VMEM((HROWS // CH, CH), jnp.int32),
            pltpu.SemaphoreType.DMA,
            pltpu.SemaphoreType.DMA,
        ],
        compiler_params=pltpu.CompilerParams(use_tc_tiling_on_sc=False,
                                             needs_layout_passes=False),
    )
    def deg(e0, e1, e2, e3, e4, zrows_i, iota3, out, acc, hist, ib, idq,
            isem, ssem):
        c = lax.axis_index("c")
        s = lax.axis_index("s")
        edges = [e0, e1, e2, e3, e4]
        ones = jnp.ones((D,), jnp.int32)

        zslice = max_slots * HROWS // NS
        pltpu.sync_copy(zrows_i.at[pl.ds(0, zslice)],
                        acc.at[pl.ds(s * zslice, zslice)])
        pltpu.sync_copy(zrows_i, hist)

        for t, (sc, s_lo, s_hi, slot) in enumerate(groups):
            n_per = n_pers[t]
            nb = n_per // HB
            assert nb % 2 == 0

            @pl.when((c == sc) & (s >= s_lo) & (s < s_hi))
            def _(t=t, s_lo=s_lo, slot=slot, nb=nb):
                p = s - s_lo
                ev = edges[t]
                pltpu.sync_copy(iota3.at[slot], idq)
                pltpu.async_copy(ev.at[p, pl.ds(0, HB)], ib.at[0], isem)

                def blk2(j2, _):
                    for sl in range(2):
                        j = j2 * 2 + sl
                        pltpu.make_async_copy(ev.at[p, pl.ds(j * HB, HB)],
                                              ib.at[sl], isem).wait()

                        @pl.when(j < nb - 1)
                        def _():
                            pltpu.async_copy(ev.at[p, pl.ds((j + 1) * HB, HB)],
                                             ib.at[1 - sl], isem)

                        def vec(k, _, sl=sl):
                            v = ib[sl, pl.ds(k * D, D)]
                            plsc.addupdate_scatter(hist, [v >> 4, v & 15], ones)
                            return 0

                        lax.fori_loop(0, HB // D, vec, 0)
                    return 0

                lax.fori_loop(0, nb // 2, blk2, 0)

        plsc.subcore_barrier()

        for t, (sc, s_lo, s_hi, slot) in enumerate(groups):
            @pl.when((c == sc) & (s >= s_lo) & (s < s_hi))
            def _(t=t):
                for j in range(HROWS // CH):
                    pltpu.async_copy(hist.at[pl.ds(j * CH, CH)],
                                     acc.at[idq.at[j]], ssem, add=True)
                pltpu.make_async_copy(hist, acc.at[pl.ds(0, HROWS)], ssem).wait()

        plsc.subcore_barrier()
        for t, (sc, s_lo, s_hi, slot) in enumerate(groups):
            @pl.when((c == sc) & (s == s_lo))
            def _(t=t, slot=slot):
                pltpu.sync_copy(acc.at[pl.ds(slot * HROWS, HROWS)], out.at[t])

    return deg


def _make_segsum(n_ch, table_rows, stage=False):
    mesh = plsc.VectorSubcoreMesh(core_axis_name="c", subcore_axis_name="s")
    per_sc = n_ch * NS * CH

    n_blk = n_ch // IBC
    nb4 = n_blk // 4
    assert n_blk % 4 == 0 and nb4 >= 1

    @functools.partial(
        pl.kernel,
        out_type=jax.ShapeDtypeStruct((NC, NJP, D), jnp.float32),
        mesh=mesh,
        scratch_types=[
            pltpu.VMEM_SHARED((NJP, D), jnp.float32),
            pltpu.VMEM((4, IBC, CH), jnp.int32),
            pltpu.VMEM((4, IBC, CH), jnp.int32),
            pltpu.VMEM((2, BLK_E, D), jnp.float32),
            pltpu.VMEM_SHARED((table_rows if stage else 1, D), jnp.float32),
        ] + [pltpu.SemaphoreType.DMA] * 8,
        compiler_params=pltpu.CompilerParams(use_tc_tiling_on_sc=False),
    )
    def segsum(table0, table1, src, dst, zrows, out, acc, idxs, idxd, rows,
               tstage, i0, i1, i2, i3, g0, g1, s0, s1):
        c = lax.axis_index("c")
        s = lax.axis_index("s")
        isem = [i0, i1, i2, i3]
        gsem = [g0, g1]
        ssem = [s0, s1]
        pltpu.sync_copy(zrows, acc.at[pl.ds(s * ROWS_PER_TILE, ROWS_PER_TILE)])
        if stage:
            @pl.when((c == 0) & (s == 0))
            def _():
                pltpu.sync_copy(table0, tstage)

            @pl.when((c == 1) & (s == 0))
            def _():
                pltpu.sync_copy(table1, tstage)
        plsc.subcore_barrier()

        cbase = s * n_ch

        def issue_idx(b, u):
            pltpu.async_copy(src.at[c, pl.ds(cbase + b * IBC, IBC)],
                             idxs.at[u], isem[u])
            pltpu.async_copy(dst.at[c, pl.ds(cbase + b * IBC, IBC)],
                             idxd.at[u], isem[u])

        def wait_idx(b, u):
            pltpu.make_async_copy(src.at[c, pl.ds(cbase + b * IBC, IBC)],
                                  idxs.at[u], isem[u]).wait()
            pltpu.make_async_copy(dst.at[c, pl.ds(cbase + b * IBC, IBC)],
                                  idxd.at[u], isem[u]).wait()

        def run(gtab):
            def scat_desc(u, sl, k):
                return pltpu.make_async_copy(rows.at[sl].at[pl.ds(k * CH, CH)],
                                             acc.at[idxd.at[u, k]], ssem[sl])

            def drain_scat(u, sl):
                for k in range(IBC):
                    scat_desc(u, sl, k).wait()

            def blk4(b4, _):
                for u in range(4):
                    b = b4 * 4 + u
                    sl = u % 2
                    u2 = (u + 2) % 4

                    if u < 2:
                        @pl.when(b4 > 0)
                        def _():
                            drain_scat(u2, sl)

                        issue_idx(b + 2, u2)
                    else:
                        drain_scat(u2, sl)

                        @pl.when(b4 < nb4 - 1)
                        def _():
                            issue_idx(b + 2, u2)

                    wait_idx(b, u)
                    gds = []
                    for k in range(IBC):
                        gds.append(pltpu.async_copy(
                            gtab.at[idxs.at[u, k]],
                            rows.at[sl].at[pl.ds(k * CH, CH)], gsem[sl]))
                    for g in gds:
                        g.wait()
                    for k in range(IBC):
                        pltpu.async_copy(rows.at[sl].at[pl.ds(k * CH, CH)],
                                         acc.at[idxd.at[u, k]], ssem[sl],
                                         add=True)
                return 0

            issue_idx(0, 0)
            issue_idx(1, 1)
            lax.fori_loop(0, nb4, blk4, 0)
            drain_scat(2, 0)
            drain_scat(3, 1)

        @pl.when(c == 0)
        def _():
            run(tstage if stage else table0)

        @pl.when(c == 1)
        def _():
            run(tstage if stage else table1)

        plsc.subcore_barrier()
        pltpu.sync_copy(acc.at[pl.ds(s * ROWS_PER_TILE, ROWS_PER_TILE)],
                        out.at[c, pl.ds(s * ROWS_PER_TILE, ROWS_PER_TILE)])

    def call(table0, table1, src2, dst2, zrows):
        assert src2.shape == (NC, per_sc) and table0.shape == (table_rows, D)
        return segsum(table0, table1, src2.reshape(NC, per_sc // CH, CH),
                      dst2.reshape(NC, per_sc // CH, CH), zrows)

    return call


def _embed_body(x, w8, bj, xw, ww, bw, dop, don, wp, wn, hj, yp, yn, hww):
    h = jnp.dot(x[...], w8[...], preferred_element_type=jnp.float32) + bj[...]
    hj[...] = h
    yp[...] = jnp.dot(h * lax.rsqrt(jnp.maximum(dop[...], 1.0)), wp[...],
                      preferred_element_type=jnp.float32)
    yn[...] = jnp.dot(h * lax.rsqrt(jnp.maximum(don[...], 1.0)), wn[...],
                      preferred_element_type=jnp.float32)
    hww[...] = jnp.dot(xw[...], ww[...], preferred_element_type=jnp.float32) + bw[...]


def _round_core(hj, aggp, aggn, sg0, sg1, dip, din, dpc, ws, wsn, bp, bn, bs):
    sip = lax.rsqrt(jnp.maximum(dip[...], 1.0))
    sin = lax.rsqrt(jnp.maximum(din[...], 1.0))
    dpr = 1.0 / jnp.maximum(dpc[...], 1.0)
    sage = (sg0[...] + sg1[...]) * dpr
    return (aggp[...] * sip + aggn[...] * sin
            + jnp.dot(hj[...], ws[...], preferred_element_type=jnp.float32)
            + jnp.dot(sage, wsn[...], preferred_element_type=jnp.float32)
            + bp[...] + bn[...] + bs[...])


def _comb1_body(hj, aggp, aggn, sg0, sg1, dip, din, dpc, dop, don,
                ws, wsn, bp, bn, bs, wp, wn, hj1, yp, yn):
    h = _round_core(hj, aggp, aggn, sg0, sg1, dip, din, dpc, ws, wsn, bp, bn, bs)
    hj1[...] = h
    yp[...] = jnp.dot(h * lax.rsqrt(jnp.maximum(dop[...], 1.0)), wp[...],
                      preferred_element_type=jnp.float32)
    yn[...] = jnp.dot(h * lax.rsqrt(jnp.maximum(don[...], 1.0)), wn[...],
                      preferred_element_type=jnp.float32)


def _comb2_body(hj, aggp, aggn, sg0, sg1, dip, din, dpc,
                ws, wsn, bp, bn, bs, hw, res):
    h = _round_core(hj, aggp, aggn, sg0, sg1, dip, din, dpc, ws, wsn, bp, bn, bs)
    res[...] = lax.dot_general(h, hw[...], (((1,), (1,)), ((), ())),
                               preferred_element_type=jnp.float32)


def _row_spec(w=D):
    return pl.BlockSpec((BLK, w), lambda i: (i, 0))


def _const_spec(shape):
    return pl.BlockSpec(shape, lambda i: (0,) * len(shape))


def kernel(x_job, x_worker, edge_precede, edge_next, proc_src, proc_dst,
           W_job, b_job, W_worker, b_worker, W_precede, b_precede,
           W_next, b_next, W_sage_self, W_sage_neigh, b_sage):
    f32, i32 = jnp.float32, jnp.int32

    zrows = jnp.zeros((ROWS_PER_TILE, D), f32)

    gc_per_sc, gc_nch = _pad_len(E)
    padg = gc_per_sc - E

    def pad_edges(src, dst, n):
        return (jnp.concatenate([src, jnp.zeros((n,), i32)]),
                jnp.concatenate([dst, jnp.full((n,), DUMMY, i32)]))

    sp, dp = pad_edges(edge_precede[0], edge_precede[1], padg)
    sn, dn = pad_edges(edge_next[0], edge_next[1], padg)
    gc_src = jnp.stack([sp, sn])
    gc_dst = jnp.stack([dp, dn])

    sg_total, _ = _pad_len(EP // NC)
    sg_nch = sg_total // (NS * CH)
    ssrc, sdst = pad_edges(proc_src, proc_dst, NC * sg_total - EP)
    SG_REP = 64
    ssrc = ssrc + NW * (jnp.arange(ssrc.shape[0], dtype=i32) & (SG_REP - 1))
    sg_src = ssrc.reshape(NC, sg_total)
    sg_dst = sdst.reshape(NC, sg_total)

    DG_GROUPS = [(0, 0, 6, 0), (0, 6, 12, 1), (1, 0, 8, 0), (1, 8, 16, 1),
                 (0, 12, 16, 2)]
    dg_arrays = [edge_precede[0], edge_precede[1], edge_next[0], edge_next[1],
                 proc_dst]
    dg_npers, dg_padded = [], []
    for t, arr in enumerate(dg_arrays):
        g = DG_GROUPS[t][2] - DG_GROUPS[t][1]
        n_per = -(--(-arr.shape[0] // g) // (2 * HB)) * (2 * HB)
        dg_npers.append(n_per)
        dg_padded.append(jnp.concatenate(
            [arr, jnp.full((g * n_per - arr.shape[0],), DUMMY, i32)]
        ).reshape(g, n_per))
    zrows_i = jnp.zeros((HROWS, D), i32)
    iota3 = (jnp.arange(HROWS, dtype=i32).reshape(HROWS // CH, CH)[None]
             + (jnp.arange(3, dtype=i32) * HROWS)[:, None, None])

    xjp = jnp.pad(x_job, ((0, NJP - NJ), (0, 1)))
    w8 = jnp.pad(W_job, ((0, 1), (0, 0)))
    xwp = jnp.pad(x_worker, ((0, 0), (0, 1)))
    ww4 = jnp.pad(W_worker, ((0, 1), (0, 0)))
    b2 = lambda b: b.reshape(1, D)

    deg_hist = _make_deg_hist(DG_GROUPS, dg_npers)
    seg_sage = _make_segsum(sg_nch, NW * 64)
    seg_gc = _make_segsum(gc_nch, NJP)

    deg5 = deg_hist(*dg_padded, zrows_i, iota3)
    degf = deg5.reshape(5, NJP, 1).astype(f32)

    embed = pl.pallas_call(
        _embed_body,
        grid=(GRID,),
        in_specs=[_row_spec(8), _const_spec((8, D)), _const_spec((1, D)),
                  _const_spec((NW, 4)), _const_spec((4, D)), _const_spec((1, D)),
                  _row_spec(1), _row_spec(1),
                  _const_spec((D, D)), _const_spec((D, D))],
        out_specs=[_row_spec(), _row_spec(), _row_spec(),
                   _const_spec((NW, D))],
        out_shape=[jax.ShapeDtypeStruct((NJP, D), f32)] * 3
        + [jax.ShapeDtypeStruct((NW, D), f32)],
    )
    hj0, yp1, yn1, hw = embed(xjp, w8, b2(b_job), xwp, ww4, b2(b_worker),
                              degf[0], degf[2], W_precede, W_next)

    hwrep = jnp.tile(hw, (64, 1))
    sage2 = seg_sage(hwrep, hwrep, sg_src, sg_dst, zrows)

    agg1 = seg_gc(yp1, yn1, gc_src, gc_dst, zrows)

    comb_in_specs = ([_row_spec()] * 5 + [_row_spec(1)] * 3
                     + [_const_spec((D, D))] * 2 + [_const_spec((1, D))] * 3)
    comb1 = pl.pallas_call(
        _comb1_body,
        grid=(GRID,),
        in_specs=([_row_spec()] * 5 + [_row_spec(1)] * 5
                  + [_const_spec((D, D))] * 2 + [_const_spec((1, D))] * 3
                  + [_const_spec((D, D))] * 2),
        out_specs=[_row_spec()] * 3,
        out_shape=[jax.ShapeDtypeStruct((NJP, D), f32)] * 3,
    )
    hj1, yp2, yn2 = comb1(hj0, agg1[0], agg1[1], sage2[0], sage2[1],
                          degf[1], degf[3], degf[4], degf[0], degf[2],
                          W_sage_self, W_sage_neigh,
                          b2(b_precede), b2(b_next), b2(b_sage),
                          W_precede, W_next)

    agg2 = seg_gc(yp2, yn2, gc_src, gc_dst, zrows)

    comb2 = pl.pallas_call(
        _comb2_body,
        grid=(GRID,),
        in_specs=comb_in_specs + [_const_spec((NW, D))],
        out_specs=pl.BlockSpec((BLK, NW), lambda i: (i, 0)),
        out_shape=jax.ShapeDtypeStruct((NJP, NW), f32),
    )
    res = comb2(hj1, agg2[0], agg2[1], sage2[0], sage2[1],
                degf[1], degf[3], degf[4],
                W_sage_self, W_sage_neigh,
                b2(b_precede), b2(b_next), b2(b_sage), hw)
    return res[:NJ]
